# Initial kernel scaffold; baseline (speedup 1.0000x reference)
import jax
import jax.numpy as jnp
from jax.experimental import pallas as pl


def kernel(inputs, embedding):
    raise NotImplementedError("write your pallas kernel here")


# speedup vs baseline: 1.0035x; 1.0035x over previous
import functools

import jax
import jax.numpy as jnp
from jax import lax
from jax.experimental import pallas as pl
from jax.experimental.pallas import tpu as pltpu
from jax.experimental.pallas import tpu_sc as plsc

_K = 8192
_DIM = 64
_TOK = 8192
_TB = 256
_CB = 4096
_CC = 0.25


def _bf16_round(v):
    return v.astype(jnp.bfloat16).astype(jnp.float32)


def _argmin_body(xn_ref, x_ref, et_ref, en_ref, idx_ref, loss_ref,
                 bvalq, bval, bidx):
    i = pl.program_id(0)
    j = pl.program_id(1)

    @pl.when((i == 0) & (j == 0))
    def _():
        loss_ref[...] = jnp.zeros((1, 1), dtype=jnp.float32)

    m = jnp.dot(x_ref[...], et_ref[...], preferred_element_type=jnp.float32)
    d = (xn_ref[...] - m) + en_ref[...]
    lmin = jnp.min(d, axis=1, keepdims=True)
    cols = lax.broadcasted_iota(jnp.int32, (_TB, _CB), 1)
    lidx = jnp.min(jnp.where(d == lmin, cols, _K), axis=1, keepdims=True) + j * _CB

    @pl.when(j == 0)
    def _():
        bvalq[...] = _bf16_round(lmin)
        bval[...] = lmin
        bidx[...] = lidx

    @pl.when(j > 0)
    def _():
        better = lmin < bvalq[...]
        bidx[...] = jnp.where(better, lidx, bidx[...])
        bvalq[...] = jnp.where(better, _bf16_round(lmin), bvalq[...])
        bval[...] = jnp.where(better, lmin, bval[...])

    @pl.when(j == pl.num_programs(1) - 1)
    def _():
        idx_ref[...] = bidx[...]
        loss_ref[...] += jnp.sum(bval[...], axis=0, keepdims=True)


def _argmin_pallas(xn, flat, emb_t, en):
    return pl.pallas_call(
        _argmin_body,
        grid=(_TOK // _TB, _K // _CB),
        in_specs=[
            pl.BlockSpec((_TB, 1), lambda i, j: (i, 0)),
            pl.BlockSpec((_TB, _DIM), lambda i, j: (i, 0)),
            pl.BlockSpec((_DIM, _CB), lambda i, j: (0, j)),
            pl.BlockSpec((1, _CB), lambda i, j: (0, j)),
        ],
        out_specs=[
            pl.BlockSpec((_TB, 1), lambda i, j: (i, 0)),
            pl.BlockSpec((1, 1), lambda i, j: (0, 0)),
        ],
        out_shape=[
            jax.ShapeDtypeStruct((_TOK, 1), jnp.int32),
            jax.ShapeDtypeStruct((1, 1), jnp.float32),
        ],
        scratch_shapes=[
            pltpu.VMEM((_TB, 1), jnp.float32),
            pltpu.VMEM((_TB, 1), jnp.float32),
            pltpu.VMEM((_TB, 1), jnp.int32),
        ],
        compiler_params=pltpu.CompilerParams(
            dimension_semantics=("arbitrary", "arbitrary")),
    )(xn, flat, emb_t, en)


_GW = 128


def _sc_gather(table_p, idx):
    info = plsc.get_sparse_core_info()
    nc, ns = info.num_cores, info.num_subcores
    nw = nc * ns
    bpw = _TOK // nw
    half = 128
    chunks = bpw // half
    mesh = plsc.VectorSubcoreMesh(core_axis_name="c", subcore_axis_name="s")

    @functools.partial(
        pl.kernel,
        mesh=mesh,
        out_type=jax.ShapeDtypeStruct((_TOK, _GW), jnp.float32),
        scratch_types=[
            pltpu.VMEM((bpw,), jnp.int32),
            pltpu.VMEM((bpw, _GW), jnp.float32),
            pltpu.SemaphoreType.DMA,
        ],
    )
    def gk(table_hbm, idx_hbm, out_hbm, idx_v, rows_v, sem):
        wid = lax.axis_index("s") * nc + lax.axis_index("c")
        base = wid * bpw
        pltpu.sync_copy(idx_hbm.at[pl.ds(base, bpw)], idx_v)
        cps = [
            pltpu.async_copy(table_hbm.at[idx_v.at[pl.ds(k * half, half)]],
                             rows_v.at[pl.ds(k * half, half)], sem)
            for k in range(chunks)
        ]
        for cp in cps:
            cp.wait()
        pltpu.sync_copy(rows_v, out_hbm.at[pl.ds(base, bpw)])

    return gk(table_p, idx)


def kernel(inputs, embedding):
    x = jnp.transpose(inputs, (0, 2, 3, 1))
    input_shape = x.shape
    flat = x.reshape(-1, _DIM)
    xn = jnp.sum(flat ** 2, axis=1, keepdims=True)
    en = jnp.sum(embedding ** 2, axis=1, keepdims=True).T
    lhs16 = (2.0 * flat).astype(jnp.bfloat16)
    emb_t16 = embedding.astype(jnp.bfloat16).T

    idx2, loss_sum = _argmin_pallas(xn, lhs16, emb_t16, en)
    idx = idx2[:, 0]

    table_p = jnp.pad(embedding, ((0, 0), (0, _GW - _DIM)))
    q = _sc_gather(table_p, idx)[:, :_DIM]
    loss = loss_sum[0, 0] * (1.0 + _CC) / (_TOK * _DIM)

    quantized = flat + (q - flat)
    quantized = quantized.reshape(input_shape)
    encodings = idx.reshape(input_shape[0:3])
    nll = jnp.ones(1, dtype=jnp.float32)
    quantized = jnp.transpose(quantized, (0, 3, 1, 2))
    return (quantized, encodings, loss, nll)


# speedup vs baseline: 1.0775x; 1.0737x over previous
import functools

import jax
import jax.numpy as jnp
from jax import lax
from jax.experimental import pallas as pl
from jax.experimental.pallas import tpu as pltpu
from jax.experimental.pallas import tpu_sc as plsc

_K = 8192
_DIM = 64
_TOK = 8192
_TB = 512
_CB = 4096
_CC = 0.25


def _bf16_round(v):
    return v.astype(jnp.bfloat16).astype(jnp.float32)


def _argmin_body(xn_ref, x_ref, et_ref, en_ref, idx_ref, loss_ref,
                 bvalq, bval, bidx):
    i = pl.program_id(0)
    j = pl.program_id(1)

    @pl.when((i == 0) & (j == 0))
    def _():
        loss_ref[...] = jnp.zeros((1, 1), dtype=jnp.float32)

    m = jnp.dot(x_ref[...], et_ref[...], preferred_element_type=jnp.float32)
    d = (xn_ref[...] - m) + en_ref[...]
    lmin = jnp.min(d, axis=1, keepdims=True)
    cols = lax.broadcasted_iota(jnp.int32, (_TB, _CB), 1)
    lidx = jnp.min(jnp.where(d == lmin, cols, _K), axis=1, keepdims=True) + j * _CB

    @pl.when(j == 0)
    def _():
        bvalq[...] = _bf16_round(lmin)
        bval[...] = lmin
        bidx[...] = lidx

    @pl.when(j > 0)
    def _():
        better = lmin < bvalq[...]
        bidx[...] = jnp.where(better, lidx, bidx[...])
        bvalq[...] = jnp.where(better, _bf16_round(lmin), bvalq[...])
        bval[...] = jnp.where(better, lmin, bval[...])

    @pl.when(j == pl.num_programs(1) - 1)
    def _():
        idx_ref[...] = bidx[...]
        loss_ref[...] += jnp.sum(bval[...], axis=0, keepdims=True)


def _argmin_pallas(xn, flat, emb_t, en):
    return pl.pallas_call(
        _argmin_body,
        grid=(_TOK // _TB, _K // _CB),
        in_specs=[
            pl.BlockSpec((_TB, 1), lambda i, j: (i, 0)),
            pl.BlockSpec((_TB, _DIM), lambda i, j: (i, 0)),
            pl.BlockSpec((_DIM, _CB), lambda i, j: (0, j)),
            pl.BlockSpec((1, _CB), lambda i, j: (0, j)),
        ],
        out_specs=[
            pl.BlockSpec((_TB, 1), lambda i, j: (i, 0)),
            pl.BlockSpec((1, 1), lambda i, j: (0, 0)),
        ],
        out_shape=[
            jax.ShapeDtypeStruct((_TOK, 1), jnp.int32),
            jax.ShapeDtypeStruct((1, 1), jnp.float32),
        ],
        scratch_shapes=[
            pltpu.VMEM((_TB, 1), jnp.float32),
            pltpu.VMEM((_TB, 1), jnp.float32),
            pltpu.VMEM((_TB, 1), jnp.int32),
        ],
        compiler_params=pltpu.CompilerParams(
            dimension_semantics=("arbitrary", "arbitrary")),
    )(xn, flat, emb_t, en)


_GW = 128


def _sc_gather(table_p, idx):
    info = plsc.get_sparse_core_info()
    nc, ns = info.num_cores, info.num_subcores
    nw = nc * ns
    bpw = _TOK // nw
    half = 128
    chunks = bpw // half
    mesh = plsc.VectorSubcoreMesh(core_axis_name="c", subcore_axis_name="s")

    @functools.partial(
        pl.kernel,
        mesh=mesh,
        out_type=jax.ShapeDtypeStruct((_TOK, _GW), jnp.float32),
        scratch_types=[
            pltpu.VMEM((bpw,), jnp.int32),
            pltpu.VMEM((bpw, _GW), jnp.float32),
            pltpu.SemaphoreType.DMA,
        ],
    )
    def gk(table_hbm, idx_hbm, out_hbm, idx_v, rows_v, sem):
        wid = lax.axis_index("s") * nc + lax.axis_index("c")
        base = wid * bpw
        pltpu.sync_copy(idx_hbm.at[pl.ds(base, bpw)], idx_v)
        cps = [
            pltpu.async_copy(table_hbm.at[idx_v.at[pl.ds(k * half, half)]],
                             rows_v.at[pl.ds(k * half, half)], sem)
            for k in range(chunks)
        ]
        for cp in cps:
            cp.wait()
        pltpu.sync_copy(rows_v, out_hbm.at[pl.ds(base, bpw)])

    return gk(table_p, idx)


def kernel(inputs, embedding):
    x = jnp.transpose(inputs, (0, 2, 3, 1))
    input_shape = x.shape
    flat = x.reshape(-1, _DIM)
    xn = jnp.sum(flat ** 2, axis=1, keepdims=True)
    en = jnp.sum(embedding ** 2, axis=1, keepdims=True).T
    lhs16 = (2.0 * flat).astype(jnp.bfloat16)
    emb_t16 = embedding.astype(jnp.bfloat16).T

    idx2, loss_sum = _argmin_pallas(xn, lhs16, emb_t16, en)
    idx = idx2[:, 0]

    table_p = jnp.pad(embedding, ((0, 0), (0, _GW - _DIM)))
    q = _sc_gather(table_p, idx)[:, :_DIM]
    loss = loss_sum[0, 0] * (1.0 + _CC) / (_TOK * _DIM)

    quantized = flat + (q - flat)
    quantized = quantized.reshape(input_shape)
    encodings = idx.reshape(input_shape[0:3])
    nll = jnp.ones(1, dtype=jnp.float32)
    quantized = jnp.transpose(quantized, (0, 3, 1, 2))
    return (quantized, encodings, loss, nll)
